# baseline (speedup 1.0000x reference)
import functools

import jax
import jax.numpy as jnp
from jax import lax
from jax.experimental import pallas as pl
from jax.experimental.pallas import tpu as pltpu
from jax.experimental.pallas import tpu_sc as plsc

_NC = 2
_NS = 16
_NW = _NC * _NS


def _sc_gather_call(emb_table, conn2d, ids, ids2):
  n = ids.shape[0]
  es = emb_table.shape[1]
  ck = conn2d.shape[1]
  assert n % _NW == 0
  per_w = n // _NW
  mesh = plsc.VectorSubcoreMesh(core_axis_name="c", subcore_axis_name="s",
                                num_cores=_NC, num_subcores=_NS)

  @functools.partial(
      pl.kernel,
      out_type=(
          jax.ShapeDtypeStruct((n, es), jnp.float32),
          jax.ShapeDtypeStruct((n, ck), jnp.int32),
      ),
      mesh=mesh,
      scratch_types=[
          pltpu.VMEM((per_w,), jnp.int32),
          pltpu.VMEM((per_w,), jnp.int32),
          pltpu.VMEM((per_w, es), jnp.float32),
          pltpu.VMEM((per_w, ck), jnp.int32),
          pltpu.SemaphoreType.DMA,
          pltpu.SemaphoreType.DMA,
      ],
  )
  def sc_kernel(emb_hbm, conn_hbm, ids_hbm, ids2_hbm, emb_out, conn_out,
                ids_v, ids2_v, rows_v, conn_v, sem_e, sem_c):
    wid = lax.axis_index("s") * _NC + lax.axis_index("c")
    base = wid * per_w
    pltpu.sync_copy(ids_hbm.at[pl.ds(base, per_w)], ids_v)
    pltpu.sync_copy(ids2_hbm.at[pl.ds(base, per_w)], ids2_v)
    cp_e = pltpu.async_copy(emb_hbm.at[ids_v], rows_v, sem_e)
    cp_c = pltpu.async_copy(conn_hbm.at[ids2_v], conn_v, sem_c)
    cp_e.wait()
    pltpu.sync_copy(rows_v, emb_out.at[pl.ds(base, per_w)])
    cp_c.wait()
    pltpu.sync_copy(conn_v, conn_out.at[pl.ds(base, per_w)])

  return sc_kernel(emb_table, conn2d, ids, ids2)


def _tc_body(n_pairs_blk, n_relp, kk,
             hemb_ref, temb_ref, hconn_ref, tconn_ref, hpar_ref, tpar_ref,
             relp_ref, wbil_ref, w1_ref, w2_ref, bsum_ref,
             hout_ref, tout_ref):
  hemb = hemb_ref[...]
  temb = temb_ref[...]
  weak = temb - hemb
  relp = relp_ref[...]
  p = jnp.dot(relp, wbil_ref[...],
              preferred_element_type=jnp.float32)
  s = lax.dot_general(weak, p, (((1,), (1,)), ((), ())),
                      preferred_element_type=jnp.float32)
  iot = lax.broadcasted_iota(jnp.int32, (1, n_relp), 1)
  w1 = w1_ref[...]
  w2 = w2_ref[...]
  bsum = bsum_ref[...]

  def one_side(conn2, par, emb, out_ref):
    conn = jnp.where(par == 1, conn2[:, 2 * kk:], conn2[:, :2 * kk])
    c = jnp.zeros((n_pairs_blk, n_relp), jnp.float32)
    for k in range(kk):
      col = conn[:, 2 * k:2 * k + 1]
      c = c + (col == iot).astype(jnp.float32)
    s_masked = jnp.where(c > 0.0, s, -jnp.inf)
    m = jnp.max(s_masked, axis=1, keepdims=True)
    e = c * jnp.exp(s_masked - m)
    z = jnp.sum(e, axis=1, keepdims=True)
    alpha = e / z
    agg = lax.dot_general(alpha, relp, (((1,), (0,)), ((), ())),
                          preferred_element_type=jnp.float32)
    h = lax.dot_general(agg, w1, (((1,), (1,)), ((), ())),
                        preferred_element_type=jnp.float32)
    h = h + lax.dot_general(emb, w2, (((1,), (1,)), ((), ())),
                            preferred_element_type=jnp.float32)
    out_ref[...] = jnp.maximum(h + bsum, 0.0)

  one_side(hconn_ref[...], hpar_ref[...], hemb, hout_ref)
  one_side(tconn_ref[...], tpar_ref[...], temb, tout_ref)


def kernel(idx, connections, emb_table, rel_table, W_bil, W1, b1, W2, b2):
  b, t = idx.shape[:2]
  kk = connections.shape[1]
  es = emb_table.shape[1]
  n_rel = rel_table.shape[0]
  n_relp = ((n_rel + 127) // 128) * 128
  n_pairs = b * t
  n_ids = 2 * n_pairs

  idx = idx.astype(jnp.int32)
  heads = idx[..., 0].reshape(-1)
  tails = idx[..., 1].reshape(-1)
  ids = jnp.concatenate([heads, tails])
  n_ent = connections.shape[0]
  assert n_ent % 2 == 0 and (4 * kk) % 128 == 0
  conn2d = connections.astype(jnp.int32).reshape(n_ent // 2, 4 * kk)
  ids2 = ids >> 1
  parity = (ids & 1).reshape(n_ids, 1)

  ht_emb, conn_rows = _sc_gather_call(emb_table, conn2d, ids, ids2)

  relp = jnp.pad(rel_table, ((0, n_relp - n_rel), (0, 0)))
  bsum = (b1 + b2).reshape(1, es)
  wbil = W_bil.reshape(es, es)

  bp = 256
  assert n_pairs % bp == 0
  nb = n_pairs // bp

  grid_spec = pl.GridSpec(
      grid=(nb,),
      in_specs=[
          pl.BlockSpec((bp, es), lambda i: (i, 0)),
          pl.BlockSpec((bp, es), lambda i: (nb + i, 0)),
          pl.BlockSpec((bp, 4 * kk), lambda i: (i, 0)),
          pl.BlockSpec((bp, 4 * kk), lambda i: (nb + i, 0)),
          pl.BlockSpec((bp, 1), lambda i: (i, 0)),
          pl.BlockSpec((bp, 1), lambda i: (nb + i, 0)),
          pl.BlockSpec((n_relp, es), lambda i: (0, 0)),
          pl.BlockSpec((es, es), lambda i: (0, 0)),
          pl.BlockSpec((es, es), lambda i: (0, 0)),
          pl.BlockSpec((es, es), lambda i: (0, 0)),
          pl.BlockSpec((1, es), lambda i: (0, 0)),
      ],
      out_specs=[
          pl.BlockSpec((bp, es), lambda i: (i, 0)),
          pl.BlockSpec((bp, es), lambda i: (i, 0)),
      ],
  )

  hout, tout = pl.pallas_call(
      functools.partial(_tc_body, bp, n_relp, kk),
      grid_spec=grid_spec,
      out_shape=[
          jax.ShapeDtypeStruct((n_pairs, es), jnp.float32),
          jax.ShapeDtypeStruct((n_pairs, es), jnp.float32),
      ],
  )(ht_emb, ht_emb, conn_rows, conn_rows, parity, parity,
    relp, wbil, W1, W2, bsum)

  h = hout.reshape(b, t, 1, es)
  tl = tout.reshape(b, t, 1, es)
  return jnp.concatenate((h, tl), axis=2)


# speedup vs baseline: 78.7588x; 78.7588x over previous
import functools

import jax
import jax.numpy as jnp
from jax import lax
from jax.experimental import pallas as pl
from jax.experimental.pallas import tpu as pltpu
from jax.experimental.pallas import tpu_sc as plsc

_NC = 2
_NS = 16
_NW = _NC * _NS
_L = 16


def _sc_gather_call(emb_table, conn_t, ids):
  n = ids.shape[0]
  es = emb_table.shape[1]
  kk = conn_t.shape[0]
  v = conn_t.shape[2]
  assert n % _NW == 0 and kk == _NW and n % _L == 0
  per_w = n // _NW
  n_chunks = n // _L
  mesh = plsc.VectorSubcoreMesh(core_axis_name="c", subcore_axis_name="s",
                                num_cores=_NC, num_subcores=_NS)

  @functools.partial(
      pl.kernel,
      out_type=(
          jax.ShapeDtypeStruct((n, es), jnp.float32),
          jax.ShapeDtypeStruct((kk, n), jnp.int32),
      ),
      mesh=mesh,
      compiler_params=pltpu.CompilerParams(needs_layout_passes=False),
      scratch_types=[
          pltpu.VMEM((n,), jnp.int32),
          pltpu.VMEM((v,), jnp.int32),
          pltpu.VMEM((n,), jnp.int32),
          pltpu.VMEM((per_w, es), jnp.float32),
          pltpu.SemaphoreType.DMA,
      ],
  )
  def sc_kernel(emb_hbm, conn_hbm, ids_hbm, emb_out, rel_out,
                ids_v, row_v, out_v, rows_v, sem_e):
    wid = lax.axis_index("s") * _NC + lax.axis_index("c")
    base = wid * per_w
    pltpu.sync_copy(ids_hbm, ids_v)
    cp_e = pltpu.async_copy(emb_hbm.at[ids_v.at[pl.ds(base, per_w)]],
                            rows_v, sem_e)
    pltpu.sync_copy(conn_hbm.at[wid, 0], row_v)

    def body(i, carry):
      idvec = ids_v[pl.ds(i * _L, _L)]
      out_v[pl.ds(i * _L, _L)] = plsc.load_gather(row_v, [idvec])
      return carry

    lax.fori_loop(0, n_chunks, body, 0, unroll=2)
    pltpu.sync_copy(out_v, rel_out.at[wid])
    cp_e.wait()
    pltpu.sync_copy(rows_v, emb_out.at[pl.ds(base, per_w)])

  return sc_kernel(emb_table, conn_t, ids)


def _tc_body(bp, n_relp, kk, es,
             hemb_ref, temb_ref, hrel_ref, trel_ref,
             relp_ref, wbil_ref, w1_ref, w2_ref, bsum_ref,
             out_ref):
  hemb = hemb_ref[...]
  temb = temb_ref[...]
  weak = temb - hemb
  relp = relp_ref[...]
  p = jnp.dot(relp, wbil_ref[...],
              preferred_element_type=jnp.float32)
  s_t = lax.dot_general(p, weak, (((1,), (1,)), ((), ())),
                        preferred_element_type=jnp.float32)
  riota = lax.broadcasted_iota(jnp.int16, (n_relp, 1), 0)
  w1 = w1_ref[...]
  w2 = w2_ref[...]
  bsum = bsum_ref[...]

  def one_side(rel_t, emb, side):
    rel16 = rel_t.astype(jnp.int16)
    c = jnp.zeros((n_relp, bp), jnp.int16)
    for k in range(kk):
      row = rel16[k:k + 1, :]
      c = c + (row == riota).astype(jnp.int16)
    cf = c.astype(jnp.float32)
    s_masked = jnp.where(c > 0, s_t, -jnp.inf)
    m = jnp.max(s_masked, axis=0, keepdims=True)
    e = cf * jnp.exp(s_masked - m)
    z = jnp.sum(e, axis=0, keepdims=True)
    alpha = e / z
    agg = lax.dot_general(alpha, relp, (((0,), (0,)), ((), ())),
                          preferred_element_type=jnp.float32)
    h = lax.dot_general(agg, w1, (((1,), (1,)), ((), ())),
                        preferred_element_type=jnp.float32)
    h = h + lax.dot_general(emb, w2, (((1,), (1,)), ((), ())),
                            preferred_element_type=jnp.float32)
    h = jnp.maximum(h + bsum, 0.0)
    out_ref[:, :, side, :] = h.reshape(bp // 128, 128, es)

  one_side(hrel_ref[...], hemb, 0)
  one_side(trel_ref[...], temb, 1)


def kernel(idx, connections, emb_table, rel_table, W_bil, W1, b1, W2, b2):
  b, t = idx.shape[:2]
  kk = connections.shape[1]
  es = emb_table.shape[1]
  n_rel = rel_table.shape[0]
  n_relp = ((n_rel + 127) // 128) * 128
  n_pairs = b * t
  n_ids = 2 * n_pairs

  idx = idx.astype(jnp.int32)
  heads = idx[..., 0].reshape(-1)
  tails = idx[..., 1].reshape(-1)
  ids = jnp.concatenate([heads, tails])
  conn_t = jnp.transpose(connections.astype(jnp.int32), (1, 2, 0))

  ht_emb, rel_t = _sc_gather_call(emb_table, conn_t, ids)

  relp = jnp.pad(rel_table, ((0, n_relp - n_rel), (0, 0)))
  bsum = (b1 + b2).reshape(1, es)
  wbil = W_bil.reshape(es, es)

  bp = 1024
  assert n_pairs % bp == 0 and bp % t == 0
  nb = n_pairs // bp
  bt = bp // t

  grid_spec = pl.GridSpec(
      grid=(nb,),
      in_specs=[
          pl.BlockSpec((bp, es), lambda i: (i, 0)),
          pl.BlockSpec((bp, es), lambda i: (nb + i, 0)),
          pl.BlockSpec((kk, bp), lambda i: (0, i)),
          pl.BlockSpec((kk, bp), lambda i: (0, nb + i)),
          pl.BlockSpec((n_relp, es), lambda i: (0, 0)),
          pl.BlockSpec((es, es), lambda i: (0, 0)),
          pl.BlockSpec((es, es), lambda i: (0, 0)),
          pl.BlockSpec((es, es), lambda i: (0, 0)),
          pl.BlockSpec((1, es), lambda i: (0, 0)),
      ],
      out_specs=[
          pl.BlockSpec((bt, t, 2, es), lambda i: (i, 0, 0, 0)),
      ],
  )

  out = pl.pallas_call(
      functools.partial(_tc_body, bp, n_relp, kk, es),
      grid_spec=grid_spec,
      out_shape=[
          jax.ShapeDtypeStruct((b, t, 2, es), jnp.float32),
      ],
  )(ht_emb, ht_emb, rel_t, rel_t, relp, wbil, W1, W2, bsum)[0]

  return out


# speedup vs baseline: 81.1138x; 1.0299x over previous
import functools

import jax
import jax.numpy as jnp
from jax import lax
from jax.experimental import pallas as pl
from jax.experimental.pallas import tpu as pltpu
from jax.experimental.pallas import tpu_sc as plsc

_NC = 2
_NS = 16
_NW = _NC * _NS
_L = 16


def _sc_gather_call(emb_table, conn_t, ids):
  n = ids.shape[0]
  es = emb_table.shape[1]
  kk = conn_t.shape[0]
  v = conn_t.shape[2]
  assert n % _NW == 0 and kk == _NW and n % _L == 0
  per_w = n // _NW
  n_chunks = n // _L
  mesh = plsc.VectorSubcoreMesh(core_axis_name="c", subcore_axis_name="s",
                                num_cores=_NC, num_subcores=_NS)

  @functools.partial(
      pl.kernel,
      out_type=(
          jax.ShapeDtypeStruct((n, es), jnp.float32),
          jax.ShapeDtypeStruct((kk, n), jnp.int32),
      ),
      mesh=mesh,
      compiler_params=pltpu.CompilerParams(needs_layout_passes=False),
      scratch_types=[
          pltpu.VMEM((n,), jnp.int32),
          pltpu.VMEM((v,), jnp.int32),
          pltpu.VMEM((n,), jnp.int32),
          pltpu.VMEM((per_w, es), jnp.float32),
          pltpu.SemaphoreType.DMA,
          pltpu.SemaphoreType.DMA,
          pltpu.SemaphoreType.DMA,
      ],
  )
  def sc_kernel(emb_hbm, conn_hbm, ids_hbm, emb_out, rel_out,
                ids_v, row_v, out_v, rows_v, sem_e, sem_r, sem_o):
    wid = lax.axis_index("s") * _NC + lax.axis_index("c")
    base = wid * per_w
    cp_row = pltpu.async_copy(conn_hbm.at[wid, 0], row_v, sem_r)
    pltpu.sync_copy(ids_hbm, ids_v)
    cp_e = pltpu.async_copy(emb_hbm.at[ids_v.at[pl.ds(base, per_w)]],
                            rows_v, sem_e)
    cp_row.wait()

    def body(i, carry):
      idvec = ids_v[pl.ds(i * _L, _L)]
      out_v[pl.ds(i * _L, _L)] = plsc.load_gather(row_v, [idvec])
      return carry

    lax.fori_loop(0, n_chunks, body, 0, unroll=2)
    cp_o = pltpu.async_copy(out_v, rel_out.at[wid], sem_o)
    cp_e.wait()
    pltpu.sync_copy(rows_v, emb_out.at[pl.ds(base, per_w)])
    cp_o.wait()

  return sc_kernel(emb_table, conn_t, ids)


def _tc_body(bp, n_relp, kk, es,
             hemb_ref, temb_ref, hrel_ref, trel_ref,
             relp_ref, wbil_ref, w1_ref, w2_ref, bsum_ref,
             out_ref):
  hemb = hemb_ref[...]
  temb = temb_ref[...]
  weak = temb - hemb
  relp = relp_ref[...]
  p = jnp.dot(relp, wbil_ref[...],
              preferred_element_type=jnp.float32)
  s_t = lax.dot_general(p, weak, (((1,), (1,)), ((), ())),
                        preferred_element_type=jnp.float32)
  riota = lax.broadcasted_iota(jnp.int16, (n_relp, 1), 0)
  w1 = w1_ref[...]
  w2 = w2_ref[...]
  bsum = bsum_ref[...]

  def one_side(rel_t, emb, side):
    rel16 = rel_t.astype(jnp.int16)
    c = jnp.zeros((n_relp, bp), jnp.int16)
    for k in range(kk):
      row = rel16[k:k + 1, :]
      c = c + (row == riota).astype(jnp.int16)
    cf = c.astype(jnp.float32)
    s_masked = jnp.where(c > 0, s_t, -jnp.inf)
    m = jnp.max(s_masked, axis=0, keepdims=True)
    e = cf * jnp.exp(s_masked - m)
    z = jnp.sum(e, axis=0, keepdims=True)
    alpha = e / z
    agg = lax.dot_general(alpha, relp, (((0,), (0,)), ((), ())),
                          preferred_element_type=jnp.float32)
    h = lax.dot_general(agg, w1, (((1,), (1,)), ((), ())),
                        preferred_element_type=jnp.float32)
    h = h + lax.dot_general(emb, w2, (((1,), (1,)), ((), ())),
                            preferred_element_type=jnp.float32)
    h = jnp.maximum(h + bsum, 0.0)
    out_ref[:, :, side, :] = h.reshape(bp // 128, 128, es)

  one_side(hrel_ref[...], hemb, 0)
  one_side(trel_ref[...], temb, 1)


def kernel(idx, connections, emb_table, rel_table, W_bil, W1, b1, W2, b2):
  b, t = idx.shape[:2]
  kk = connections.shape[1]
  es = emb_table.shape[1]
  n_rel = rel_table.shape[0]
  n_relp = ((n_rel + 127) // 128) * 128
  n_pairs = b * t
  n_ids = 2 * n_pairs

  idx = idx.astype(jnp.int32)
  heads = idx[..., 0].reshape(-1)
  tails = idx[..., 1].reshape(-1)
  ids = jnp.concatenate([heads, tails])
  conn_t = jnp.transpose(connections.astype(jnp.int32), (1, 2, 0))

  ht_emb, rel_t = _sc_gather_call(emb_table, conn_t, ids)

  relp = jnp.pad(rel_table, ((0, n_relp - n_rel), (0, 0)))
  bsum = (b1 + b2).reshape(1, es)
  wbil = W_bil.reshape(es, es)

  bp = 1024
  assert n_pairs % bp == 0 and bp % t == 0
  nb = n_pairs // bp
  bt = bp // t

  grid_spec = pl.GridSpec(
      grid=(nb,),
      in_specs=[
          pl.BlockSpec((bp, es), lambda i: (i, 0)),
          pl.BlockSpec((bp, es), lambda i: (nb + i, 0)),
          pl.BlockSpec((kk, bp), lambda i: (0, i)),
          pl.BlockSpec((kk, bp), lambda i: (0, nb + i)),
          pl.BlockSpec((n_relp, es), lambda i: (0, 0)),
          pl.BlockSpec((es, es), lambda i: (0, 0)),
          pl.BlockSpec((es, es), lambda i: (0, 0)),
          pl.BlockSpec((es, es), lambda i: (0, 0)),
          pl.BlockSpec((1, es), lambda i: (0, 0)),
      ],
      out_specs=[
          pl.BlockSpec((bt, t, 2, es), lambda i: (i, 0, 0, 0)),
      ],
  )

  out = pl.pallas_call(
      functools.partial(_tc_body, bp, n_relp, kk, es),
      grid_spec=grid_spec,
      out_shape=[
          jax.ShapeDtypeStruct((b, t, 2, es), jnp.float32),
      ],
  )(ht_emb, ht_emb, rel_t, rel_t, relp, wbil, W1, W2, bsum)[0]

  return out
